# baseline (speedup 1.0000x reference)
import functools

import jax
import jax.numpy as jnp
from jax import lax
from jax.experimental import pallas as pl
from jax.experimental.pallas import tpu as pltpu
from jax.experimental.pallas import tpu_sc as plsc

BS = 4
N = 4096
K = 16
EPS = 1e-5
CP = 32
R = 256
NB = N // R


def _topk_body(xt_rows_ref, xo_ref, idx_ref):
    b = pl.program_id(0)
    rows = xt_rows_ref[0]
    xo = xo_ref[0]
    xx_r = jnp.sum(rows * rows, axis=1, keepdims=True)
    xx_a = jnp.sum(xo * xo, axis=0, keepdims=True)
    dot = lax.dot_general(rows, xo, (((1,), (0,)), ((), ())),
                          preferred_element_type=jnp.float32)
    d = 2.0 * dot - xx_r - xx_a
    colid_f = lax.broadcasted_iota(jnp.int32, (R, N), 1).astype(jnp.float32)
    neg_inf = jnp.float32(-jnp.inf)
    cols = []
    for _ in range(K):
        v, i = d, colid_f
        w = N
        while w > 1:
            h = w // 2
            take = v[:, h:w] > v[:, :h]
            v = jnp.where(take, v[:, h:w], v[:, :h])
            i = jnp.where(take, i[:, h:w], i[:, :h])
            w = h
        cols.append(i)
        d = jnp.where(colid_f == i, neg_inf, d)
    idx = jnp.concatenate(cols, axis=1).astype(jnp.int32)
    idx_ref[0] = idx + b * N


def _dist_topk(xt, xo):
    return pl.pallas_call(
        _topk_body,
        grid=(BS, NB),
        in_specs=[
            pl.BlockSpec((1, R, CP), lambda b, i: (b, i, 0)),
            pl.BlockSpec((1, CP, N), lambda b, i: (b, 0, 0)),
        ],
        out_specs=pl.BlockSpec((1, R, K), lambda b, i: (b, i, 0)),
        out_shape=jax.ShapeDtypeStruct((BS, N, K), jnp.int32),
    )(xt, xo)


_CH = 128


def _gather_rows(table, idx):
    info = plsc.get_sparse_core_info()
    nw = info.num_cores * info.num_subcores
    b_idx = idx.shape[0]
    b_per_w = b_idx // nw
    n_ch = b_per_w // _CH
    mesh = plsc.VectorSubcoreMesh(core_axis_name="c", subcore_axis_name="s")
    num_cores = info.num_cores

    @functools.partial(
        pl.kernel,
        mesh=mesh,
        compiler_params=pltpu.CompilerParams(use_tc_tiling_on_sc=False),
        out_type=jax.ShapeDtypeStruct((b_idx, CP), jnp.float32),
        scratch_types=[
            pltpu.VMEM((_CH,), jnp.int32),
            pltpu.VMEM((_CH, CP), jnp.float32),
            pltpu.SemaphoreType.DMA,
        ],
    )
    def k(idx_hbm, table_hbm, out_hbm, idx_v, rows_v, sem):
        c = lax.axis_index("c")
        s = lax.axis_index("s")
        base = (s * num_cores + c) * b_per_w

        def body(i, carry):
            off = base + i * _CH
            pltpu.sync_copy(idx_hbm.at[pl.ds(off, _CH)], idx_v)
            pltpu.async_copy(table_hbm.at[idx_v], rows_v, sem).wait()
            pltpu.sync_copy(rows_v, out_hbm.at[pl.ds(off, _CH)])
            return carry

        lax.fori_loop(0, n_ch, body, 0)

    return k(idx, table)


def _layer(x, w_ref, b_ref, s_ref, bt_ref):
    y = lax.dot_general(x, w_ref[...], (((1,), (0,)), ((), ())),
                        preferred_element_type=jnp.float32) + b_ref[...]
    return jnp.maximum(y * s_ref[...] + bt_ref[...], 0.0)


def _edge_body(g_ref, cent_ref, w1_ref, b1_ref, s1_ref, bt1_ref,
               w2_ref, b2_ref, s2_ref, bt2_ref, out_ref):
    G = g_ref[0]
    C = cent_ref[0]
    Crep = jnp.reshape(
        jnp.broadcast_to(C[:, None, :], (R, K, CP)), (R * K, CP))
    e = jnp.concatenate([G - Crep, Crep], axis=1)
    a1 = _layer(e, w1_ref, b1_ref, s1_ref, bt1_ref)
    a2 = _layer(a1, w2_ref, b2_ref, s2_ref, bt2_ref)
    c2 = a2.shape[1]
    out_ref[0] = jnp.max(jnp.reshape(a2, (R, K, c2)), axis=1)


def _vec_specs(c1, c2):
    return [
        pl.BlockSpec((2 * CP, c1), lambda b, i: (0, 0)),
        pl.BlockSpec((1, c1), lambda b, i: (0, 0)),
        pl.BlockSpec((1, c1), lambda b, i: (0, 0)),
        pl.BlockSpec((1, c1), lambda b, i: (0, 0)),
        pl.BlockSpec((c1, c2), lambda b, i: (0, 0)),
        pl.BlockSpec((1, c2), lambda b, i: (0, 0)),
        pl.BlockSpec((1, c2), lambda b, i: (0, 0)),
        pl.BlockSpec((1, c2), lambda b, i: (0, 0)),
    ]


def _edge_mlp(g, cent, lp):
    c1 = lp[0].shape[1]
    c2 = lp[4].shape[1]
    return pl.pallas_call(
        _edge_body,
        grid=(BS, NB),
        in_specs=[
            pl.BlockSpec((1, R * K, CP), lambda b, i: (b, i, 0)),
            pl.BlockSpec((1, R, CP), lambda b, i: (b, i, 0)),
        ] + _vec_specs(c1, c2),
        out_specs=pl.BlockSpec((1, R, c2), lambda b, i: (b, i, 0)),
        out_shape=jax.ShapeDtypeStruct((BS, N, c2), jnp.float32),
    )(g, cent, *lp)


def _head_body(x1_ref, x2_ref, w1_ref, b1_ref, s1_ref, bt1_ref,
               w2_ref, b2_ref, s2_ref, bt2_ref, out_ref):
    cc = jnp.concatenate([x1_ref[0][:, :19], x2_ref[0]], axis=1)
    y1 = _layer(cc, w1_ref, b1_ref, s1_ref, bt1_ref)
    y2 = _layer(y1, w2_ref, b2_ref, s2_ref, bt2_ref)
    out_ref[0] = y2


def _head_mlp(x1, x2, lp):
    c0, c1 = lp[0].shape
    c2 = lp[4].shape[1]
    specs = [
        pl.BlockSpec((c0, c1), lambda b, i: (0, 0)),
        pl.BlockSpec((1, c1), lambda b, i: (0, 0)),
        pl.BlockSpec((1, c1), lambda b, i: (0, 0)),
        pl.BlockSpec((1, c1), lambda b, i: (0, 0)),
        pl.BlockSpec((c1, c2), lambda b, i: (0, 0)),
        pl.BlockSpec((1, c2), lambda b, i: (0, 0)),
        pl.BlockSpec((1, c2), lambda b, i: (0, 0)),
        pl.BlockSpec((1, c2), lambda b, i: (0, 0)),
    ]
    return pl.pallas_call(
        _head_body,
        grid=(BS, NB),
        in_specs=[
            pl.BlockSpec((1, R, CP), lambda b, i: (b, i, 0)),
            pl.BlockSpec((1, R, 128), lambda b, i: (b, i, 0)),
        ] + specs,
        out_specs=pl.BlockSpec((1, R, c2), lambda b, i: (b, i, 0)),
        out_shape=jax.ShapeDtypeStruct((BS, N, c2), jnp.float32),
    )(x1, x2, *lp)


def _bn_vecs(p, c_pad):
    c = p['b'].shape[0]
    s = p['g'] / jnp.sqrt(1.0 + EPS)
    b = jnp.zeros((1, c_pad), jnp.float32).at[:, :c].set(p['b'][None, :])
    sv = jnp.ones((1, c_pad), jnp.float32).at[:, :c].set(s[None, :])
    bt = jnp.zeros((1, c_pad), jnp.float32).at[:, :c].set(p['beta'][None, :])
    return b, sv, bt


def _split_w1(w, c):
    c1 = w.shape[0]
    out = jnp.zeros((2 * CP, c1), jnp.float32)
    out = out.at[:c].set(w[:, :c].T)
    out = out.at[CP:CP + c].set(w[:, c:].T)
    return out


def _prep_edge(layers, c, c2_pad):
    p1, p2 = layers
    c2 = p2['W'].shape[0]
    w1 = _split_w1(p1['W'], c)
    b1, s1, bt1 = _bn_vecs(p1, p1['W'].shape[0])
    w2 = jnp.zeros((w1.shape[1], c2_pad), jnp.float32).at[:, :c2].set(
        p2['W'].T)
    b2, s2, bt2 = _bn_vecs(p2, c2_pad)
    return w1, b1, s1, bt1, w2, b2, s2, bt2


def kernel(x, params):
    xo1 = jnp.zeros((BS, CP, N), jnp.float32).at[:, :3].set(x)
    xt1 = jnp.transpose(xo1, (0, 2, 1))

    lp1 = _prep_edge(params['conv1'], 3, CP)
    lp2 = _prep_edge(params['conv2'], 19, 128)

    idx1 = _dist_topk(xt1, xo1)
    g1 = _gather_rows(xt1.reshape(BS * N, CP), idx1.reshape(BS * N * K))
    x1t = _edge_mlp(g1.reshape(BS, N * K, CP), xt1, lp1)

    xo2 = jnp.transpose(x1t, (0, 2, 1))
    idx2 = _dist_topk(x1t, xo2)
    g2 = _gather_rows(x1t.reshape(BS * N, CP), idx2.reshape(BS * N * K))
    x2t = _edge_mlp(g2.reshape(BS, N * K, CP), x1t, lp2)

    p31, p32 = params['conv3']
    lp3 = (p31['W'].T, *_bn_vecs(p31, 137), p32['W'].T, *_bn_vecs(p32, 128))
    yt = _head_mlp(x1t, x2t, lp3)
    return jnp.transpose(yt, (0, 2, 1))


# speedup vs baseline: 1.4503x; 1.4503x over previous
import functools

import jax
import jax.numpy as jnp
from jax import lax
from jax.experimental import pallas as pl
from jax.experimental.pallas import tpu as pltpu
from jax.experimental.pallas import tpu_sc as plsc

BS = 4
N = 4096
K = 16
EPS = 1e-5
CP = 32
R = 256
NB = N // R


def _topk_body(xt_rows_ref, xo_ref, idx_ref):
    b = pl.program_id(0)
    rows = xt_rows_ref[0]
    xo = xo_ref[0]
    xx_r = jnp.sum(rows * rows, axis=1, keepdims=True)
    xx_a = jnp.sum(xo * xo, axis=0, keepdims=True)
    dot = lax.dot_general(rows, xo, (((1,), (0,)), ((), ())),
                          preferred_element_type=jnp.float32)
    d = 2.0 * dot - xx_r - xx_a
    colid_f = lax.broadcasted_iota(jnp.int32, (R, N), 1).astype(jnp.float32)
    neg_inf = jnp.float32(-jnp.inf)
    big = jnp.float32(N)
    cols = []
    for _ in range(K):
        m = jnp.max(d, axis=1, keepdims=True)
        sel = jnp.where(d == m, colid_f, big)
        i = jnp.min(sel, axis=1, keepdims=True)
        cols.append(i)
        d = jnp.where(colid_f == i, neg_inf, d)
    idx = jnp.concatenate(cols, axis=1).astype(jnp.int32)
    idx_ref[0] = idx + b * N


def _dist_topk(xt, xo):
    return pl.pallas_call(
        _topk_body,
        grid=(BS, NB),
        in_specs=[
            pl.BlockSpec((1, R, CP), lambda b, i: (b, i, 0)),
            pl.BlockSpec((1, CP, N), lambda b, i: (b, 0, 0)),
        ],
        out_specs=pl.BlockSpec((1, R, K), lambda b, i: (b, i, 0)),
        out_shape=jax.ShapeDtypeStruct((BS, N, K), jnp.int32),
    )(xt, xo)


_CH = 128


def _gather_rows(table, idx):
    info = plsc.get_sparse_core_info()
    nw = info.num_cores * info.num_subcores
    b_idx = idx.shape[0]
    b_per_w = b_idx // nw
    n_ch = b_per_w // _CH
    mesh = plsc.VectorSubcoreMesh(core_axis_name="c", subcore_axis_name="s")
    num_cores = info.num_cores

    @functools.partial(
        pl.kernel,
        mesh=mesh,
        compiler_params=pltpu.CompilerParams(use_tc_tiling_on_sc=False),
        out_type=jax.ShapeDtypeStruct((b_idx, CP), jnp.float32),
        scratch_types=[
            pltpu.VMEM((_CH,), jnp.int32),
            pltpu.VMEM((_CH, CP), jnp.float32),
            pltpu.SemaphoreType.DMA,
        ],
    )
    def k(idx_hbm, table_hbm, out_hbm, idx_v, rows_v, sem):
        c = lax.axis_index("c")
        s = lax.axis_index("s")
        base = (s * num_cores + c) * b_per_w

        def body(i, carry):
            off = base + i * _CH
            pltpu.sync_copy(idx_hbm.at[pl.ds(off, _CH)], idx_v)
            pltpu.async_copy(table_hbm.at[idx_v], rows_v, sem).wait()
            pltpu.sync_copy(rows_v, out_hbm.at[pl.ds(off, _CH)])
            return carry

        lax.fori_loop(0, n_ch, body, 0)

    return k(idx, table)


def _layer(x, w_ref, b_ref, s_ref, bt_ref):
    y = lax.dot_general(x, w_ref[...], (((1,), (0,)), ((), ())),
                        preferred_element_type=jnp.float32) + b_ref[...]
    return jnp.maximum(y * s_ref[...] + bt_ref[...], 0.0)


def _edge_body(g_ref, cent_ref, w1_ref, b1_ref, s1_ref, bt1_ref,
               w2_ref, b2_ref, s2_ref, bt2_ref, out_ref):
    G = g_ref[0]
    C = cent_ref[0]
    Crep = jnp.reshape(
        jnp.broadcast_to(C[:, None, :], (R, K, CP)), (R * K, CP))
    e = jnp.concatenate([G - Crep, Crep], axis=1)
    a1 = _layer(e, w1_ref, b1_ref, s1_ref, bt1_ref)
    a2 = _layer(a1, w2_ref, b2_ref, s2_ref, bt2_ref)
    c2 = a2.shape[1]
    out_ref[0] = jnp.max(jnp.reshape(a2, (R, K, c2)), axis=1)


def _vec_specs(c1, c2):
    return [
        pl.BlockSpec((2 * CP, c1), lambda b, i: (0, 0)),
        pl.BlockSpec((1, c1), lambda b, i: (0, 0)),
        pl.BlockSpec((1, c1), lambda b, i: (0, 0)),
        pl.BlockSpec((1, c1), lambda b, i: (0, 0)),
        pl.BlockSpec((c1, c2), lambda b, i: (0, 0)),
        pl.BlockSpec((1, c2), lambda b, i: (0, 0)),
        pl.BlockSpec((1, c2), lambda b, i: (0, 0)),
        pl.BlockSpec((1, c2), lambda b, i: (0, 0)),
    ]


def _edge_mlp(g, cent, lp):
    c1 = lp[0].shape[1]
    c2 = lp[4].shape[1]
    return pl.pallas_call(
        _edge_body,
        grid=(BS, NB),
        in_specs=[
            pl.BlockSpec((1, R * K, CP), lambda b, i: (b, i, 0)),
            pl.BlockSpec((1, R, CP), lambda b, i: (b, i, 0)),
        ] + _vec_specs(c1, c2),
        out_specs=pl.BlockSpec((1, R, c2), lambda b, i: (b, i, 0)),
        out_shape=jax.ShapeDtypeStruct((BS, N, c2), jnp.float32),
    )(g, cent, *lp)


def _head_body(x1_ref, x2_ref, w1_ref, b1_ref, s1_ref, bt1_ref,
               w2_ref, b2_ref, s2_ref, bt2_ref, out_ref):
    cc = jnp.concatenate([x1_ref[0][:, :19], x2_ref[0]], axis=1)
    y1 = _layer(cc, w1_ref, b1_ref, s1_ref, bt1_ref)
    y2 = _layer(y1, w2_ref, b2_ref, s2_ref, bt2_ref)
    out_ref[0] = y2


def _head_mlp(x1, x2, lp):
    c0, c1 = lp[0].shape
    c2 = lp[4].shape[1]
    specs = [
        pl.BlockSpec((c0, c1), lambda b, i: (0, 0)),
        pl.BlockSpec((1, c1), lambda b, i: (0, 0)),
        pl.BlockSpec((1, c1), lambda b, i: (0, 0)),
        pl.BlockSpec((1, c1), lambda b, i: (0, 0)),
        pl.BlockSpec((c1, c2), lambda b, i: (0, 0)),
        pl.BlockSpec((1, c2), lambda b, i: (0, 0)),
        pl.BlockSpec((1, c2), lambda b, i: (0, 0)),
        pl.BlockSpec((1, c2), lambda b, i: (0, 0)),
    ]
    return pl.pallas_call(
        _head_body,
        grid=(BS, NB),
        in_specs=[
            pl.BlockSpec((1, R, CP), lambda b, i: (b, i, 0)),
            pl.BlockSpec((1, R, 128), lambda b, i: (b, i, 0)),
        ] + specs,
        out_specs=pl.BlockSpec((1, R, c2), lambda b, i: (b, i, 0)),
        out_shape=jax.ShapeDtypeStruct((BS, N, c2), jnp.float32),
    )(x1, x2, *lp)


def _bn_vecs(p, c_pad):
    c = p['b'].shape[0]
    s = p['g'] / jnp.sqrt(1.0 + EPS)
    b = jnp.zeros((1, c_pad), jnp.float32).at[:, :c].set(p['b'][None, :])
    sv = jnp.ones((1, c_pad), jnp.float32).at[:, :c].set(s[None, :])
    bt = jnp.zeros((1, c_pad), jnp.float32).at[:, :c].set(p['beta'][None, :])
    return b, sv, bt


def _split_w1(w, c):
    c1 = w.shape[0]
    out = jnp.zeros((2 * CP, c1), jnp.float32)
    out = out.at[:c].set(w[:, :c].T)
    out = out.at[CP:CP + c].set(w[:, c:].T)
    return out


def _prep_edge(layers, c, c2_pad):
    p1, p2 = layers
    c2 = p2['W'].shape[0]
    w1 = _split_w1(p1['W'], c)
    b1, s1, bt1 = _bn_vecs(p1, p1['W'].shape[0])
    w2 = jnp.zeros((w1.shape[1], c2_pad), jnp.float32).at[:, :c2].set(
        p2['W'].T)
    b2, s2, bt2 = _bn_vecs(p2, c2_pad)
    return w1, b1, s1, bt1, w2, b2, s2, bt2


def kernel(x, params):
    xo1 = jnp.zeros((BS, CP, N), jnp.float32).at[:, :3].set(x)
    xt1 = jnp.transpose(xo1, (0, 2, 1))

    lp1 = _prep_edge(params['conv1'], 3, CP)
    lp2 = _prep_edge(params['conv2'], 19, 128)

    idx1 = _dist_topk(xt1, xo1)
    g1 = _gather_rows(xt1.reshape(BS * N, CP), idx1.reshape(BS * N * K))
    x1t = _edge_mlp(g1.reshape(BS, N * K, CP), xt1, lp1)

    xo2 = jnp.transpose(x1t, (0, 2, 1))
    idx2 = _dist_topk(x1t, xo2)
    g2 = _gather_rows(x1t.reshape(BS * N, CP), idx2.reshape(BS * N * K))
    x2t = _edge_mlp(g2.reshape(BS, N * K, CP), x1t, lp2)

    p31, p32 = params['conv3']
    lp3 = (p31['W'].T, *_bn_vecs(p31, 137), p32['W'].T, *_bn_vecs(p32, 128))
    yt = _head_mlp(x1t, x2t, lp3)
    return jnp.transpose(yt, (0, 2, 1))
